# baseline (speedup 1.0000x reference)
import functools

import jax
import jax.numpy as jnp
from jax.experimental import pallas as pl
from jax.experimental.pallas import tpu as pltpu

_B, _T, _D, _K, _S = 4, 2048, 1024, 64, 8
_TT = 512


def _router_body(x_ref, mu_ref, ls_ref, eps_ref, out_ref, s_acc):
    b = pl.program_id(0)
    t = pl.program_id(1)
    nt = pl.num_programs(1)

    @pl.when((b == 0) & (t == 0))
    def _init():
        s_acc[...] = jnp.zeros_like(s_acc)

    x = x_ref[0]
    mu = mu_ref[...]
    dot = jax.lax.dot_general(
        x, mu, (((1,), (1,)), ((), ())),
        preferred_element_type=jnp.float32,
        precision=jax.lax.Precision.HIGHEST)
    x_norm = jnp.sqrt(jnp.sum(x * x, axis=1, keepdims=True))
    mu_norm = jnp.sqrt(jnp.sum(mu * mu, axis=1))
    cos = dot / jnp.maximum(x_norm * mu_norm[None, :], 1e-8)
    m = jnp.max(cos, axis=1, keepdims=True)
    e = jnp.exp(cos - m)
    p = e / jnp.sum(e, axis=1, keepdims=True)
    partial = jnp.sum(p, axis=0, keepdims=True)
    row = jax.lax.broadcasted_iota(jnp.int32, (_B, 1), 0) == b
    s_acc[...] += jnp.where(row, partial, 0.0)

    @pl.when((b == _B - 1) & (t == nt - 1))
    def _finish():
        s = s_acc[...]
        sm = jnp.max(s, axis=1, keepdims=True)
        se = jnp.exp(s - sm)
        r = se / jnp.sum(se, axis=1, keepdims=True)
        iota_k = jax.lax.broadcasted_iota(jnp.int32, (_B, _K), 1)
        rr = r
        for j in range(_S):
            mj = jnp.max(rr, axis=1, keepdims=True)
            idxj = jnp.min(jnp.where(rr == mj, iota_k, _K), axis=1,
                           keepdims=True)
            oh = (iota_k == idxj).astype(jnp.float32)
            mu_j = jax.lax.dot_general(
                oh, mu_ref[...], (((1,), (0,)), ((), ())),
                preferred_element_type=jnp.float32,
                precision=jax.lax.Precision.HIGHEST)
            ls_j = jax.lax.dot_general(
                oh, ls_ref[...], (((1,), (0,)), ((), ())),
                preferred_element_type=jnp.float32,
                precision=jax.lax.Precision.HIGHEST)
            out_ref[:, j, :] = mu_j + jnp.exp(ls_j) * eps_ref[:, j, :]
            rr = jnp.where(oh > 0, -1.0, rr)


@functools.partial(jax.jit, static_argnames=())
def _run(x, mu, log_sigma, eps):
    nt = _T // _TT
    return pl.pallas_call(
        _router_body,
        grid=(_B, nt),
        in_specs=[
            pl.BlockSpec((1, _TT, _D), lambda b, t: (b, t, 0)),
            pl.BlockSpec((_K, _D), lambda b, t: (0, 0)),
            pl.BlockSpec((_K, _D), lambda b, t: (0, 0)),
            pl.BlockSpec((_B, _S, _D), lambda b, t: (0, 0, 0)),
        ],
        out_specs=pl.BlockSpec((_B, _S, _D), lambda b, t: (0, 0, 0)),
        out_shape=jax.ShapeDtypeStruct((_B, _S, _D), jnp.float32),
        scratch_shapes=[pltpu.VMEM((_B, _K), jnp.float32)],
        compiler_params=pltpu.CompilerParams(
            dimension_semantics=("arbitrary", "arbitrary")),
    )(x, mu, log_sigma, eps)


def kernel(x, mu, log_sigma, n_slots):
    eps = jax.random.normal(jax.random.key(42), (_B, _S, _D), jnp.float32)
    return _run(x, mu, log_sigma, eps)


# speedup vs baseline: 1.6581x; 1.6581x over previous
import functools

import jax
import jax.numpy as jnp
from jax.experimental import pallas as pl
from jax.experimental.pallas import tpu as pltpu

_B, _T, _D, _K, _S = 4, 2048, 1024, 64, 8
_TT = 512


def _router_body(x_ref, mu_ref, ls_ref, eps_ref, out_ref, s_acc):
    b = pl.program_id(0)
    t = pl.program_id(1)
    nt = pl.num_programs(1)

    @pl.when((b == 0) & (t == 0))
    def _init():
        s_acc[...] = jnp.zeros_like(s_acc)

    x = x_ref[0]
    mu = mu_ref[...]
    dot = jax.lax.dot_general(
        x, mu, (((1,), (1,)), ((), ())),
        preferred_element_type=jnp.float32)
    x_norm = jnp.sqrt(jnp.sum(x * x, axis=1, keepdims=True))
    mu_norm = jnp.sqrt(jnp.sum(mu * mu, axis=1))
    cos = dot / jnp.maximum(x_norm * mu_norm[None, :], 1e-8)
    m = jnp.max(cos, axis=1, keepdims=True)
    e = jnp.exp(cos - m)
    p = e / jnp.sum(e, axis=1, keepdims=True)
    partial = jnp.sum(p, axis=0, keepdims=True)
    row = jax.lax.broadcasted_iota(jnp.int32, (_B, 1), 0) == b
    s_acc[...] += jnp.where(row, partial, 0.0)

    @pl.when((b == _B - 1) & (t == nt - 1))
    def _finish():
        s = s_acc[...]
        sm = jnp.max(s, axis=1, keepdims=True)
        se = jnp.exp(s - sm)
        r = se / jnp.sum(se, axis=1, keepdims=True)
        iota_k = jax.lax.broadcasted_iota(jnp.int32, (_B, _K), 1)
        rr = r
        for j in range(_S):
            mj = jnp.max(rr, axis=1, keepdims=True)
            idxj = jnp.min(jnp.where(rr == mj, iota_k, _K), axis=1,
                           keepdims=True)
            oh = (iota_k == idxj).astype(jnp.float32)
            mu_j = jax.lax.dot_general(
                oh, mu_ref[...], (((1,), (0,)), ((), ())),
                preferred_element_type=jnp.float32,
                precision=jax.lax.Precision.HIGHEST)
            ls_j = jax.lax.dot_general(
                oh, ls_ref[...], (((1,), (0,)), ((), ())),
                preferred_element_type=jnp.float32,
                precision=jax.lax.Precision.HIGHEST)
            out_ref[:, j, :] = mu_j + jnp.exp(ls_j) * eps_ref[:, j, :]
            rr = jnp.where(oh > 0, -1.0, rr)


@functools.partial(jax.jit, static_argnames=())
def _run(x, mu, log_sigma, eps):
    nt = _T // _TT
    return pl.pallas_call(
        _router_body,
        grid=(_B, nt),
        in_specs=[
            pl.BlockSpec((1, _TT, _D), lambda b, t: (b, t, 0)),
            pl.BlockSpec((_K, _D), lambda b, t: (0, 0)),
            pl.BlockSpec((_K, _D), lambda b, t: (0, 0)),
            pl.BlockSpec((_B, _S, _D), lambda b, t: (0, 0, 0)),
        ],
        out_specs=pl.BlockSpec((_B, _S, _D), lambda b, t: (0, 0, 0)),
        out_shape=jax.ShapeDtypeStruct((_B, _S, _D), jnp.float32),
        scratch_shapes=[pltpu.VMEM((_B, _K), jnp.float32)],
        compiler_params=pltpu.CompilerParams(
            dimension_semantics=("arbitrary", "arbitrary")),
    )(x, mu, log_sigma, eps)


def kernel(x, mu, log_sigma, n_slots):
    eps = jax.random.normal(jax.random.key(42), (_B, _S, _D), jnp.float32)
    return _run(x, mu, log_sigma, eps)


# speedup vs baseline: 2.1004x; 1.2667x over previous
import functools

import jax
import jax.numpy as jnp
from jax.experimental import pallas as pl
from jax.experimental.pallas import tpu as pltpu

_B, _T, _D, _K, _S = 4, 2048, 1024, 64, 8
_TT = 1024


def _router_body(x_ref, mu_ref, ls_ref, eps_ref, out_ref, s_acc):
    b = pl.program_id(0)
    t = pl.program_id(1)
    nt = pl.num_programs(1)

    @pl.when((b == 0) & (t == 0))
    def _init():
        s_acc[...] = jnp.zeros_like(s_acc)

    x = x_ref[0]
    mu = mu_ref[...]
    dot = jax.lax.dot_general(
        x, mu, (((1,), (1,)), ((), ())),
        preferred_element_type=jnp.float32)
    x_norm = jnp.sqrt(jnp.sum(x * x, axis=1, keepdims=True))
    mu_norm = jnp.sqrt(jnp.sum(mu * mu, axis=1))
    cos = dot / jnp.maximum(x_norm * mu_norm[None, :], 1e-8)
    m = jnp.max(cos, axis=1, keepdims=True)
    e = jnp.exp(cos - m)
    p = e / jnp.sum(e, axis=1, keepdims=True)
    partial = jnp.sum(p, axis=0, keepdims=True)
    row = jax.lax.broadcasted_iota(jnp.int32, (_B, 1), 0) == b
    s_acc[...] += jnp.where(row, partial, 0.0)

    @pl.when((b == _B - 1) & (t == nt - 1))
    def _finish():
        s = s_acc[...]
        sm = jnp.max(s, axis=1, keepdims=True)
        se = jnp.exp(s - sm)
        r = se / jnp.sum(se, axis=1, keepdims=True)
        iota_k = jax.lax.broadcasted_iota(jnp.int32, (_B, _K), 1)
        rr = r
        for j in range(_S):
            mj = jnp.max(rr, axis=1, keepdims=True)
            idxj = jnp.min(jnp.where(rr == mj, iota_k, _K), axis=1,
                           keepdims=True)
            oh = (iota_k == idxj).astype(jnp.float32)
            mu_j = jax.lax.dot_general(
                oh, mu_ref[...], (((1,), (0,)), ((), ())),
                preferred_element_type=jnp.float32,
                precision=jax.lax.Precision.HIGHEST)
            ls_j = jax.lax.dot_general(
                oh, ls_ref[...], (((1,), (0,)), ((), ())),
                preferred_element_type=jnp.float32,
                precision=jax.lax.Precision.HIGHEST)
            out_ref[:, j, :] = mu_j + jnp.exp(ls_j) * eps_ref[:, j, :]
            rr = jnp.where(oh > 0, -1.0, rr)


@functools.partial(jax.jit, static_argnames=())
def _run(x, mu, log_sigma, eps):
    nt = _T // _TT
    return pl.pallas_call(
        _router_body,
        grid=(_B, nt),
        in_specs=[
            pl.BlockSpec((1, _TT, _D), lambda b, t: (b, t, 0)),
            pl.BlockSpec((_K, _D), lambda b, t: (0, 0)),
            pl.BlockSpec((_K, _D), lambda b, t: (0, 0)),
            pl.BlockSpec((_B, _S, _D), lambda b, t: (0, 0, 0)),
        ],
        out_specs=pl.BlockSpec((_B, _S, _D), lambda b, t: (0, 0, 0)),
        out_shape=jax.ShapeDtypeStruct((_B, _S, _D), jnp.float32),
        scratch_shapes=[pltpu.VMEM((_B, _K), jnp.float32)],
        compiler_params=pltpu.CompilerParams(
            dimension_semantics=("arbitrary", "arbitrary")),
    )(x, mu, log_sigma, eps)


def kernel(x, mu, log_sigma, n_slots):
    eps = jax.random.normal(jax.random.key(42), (_B, _S, _D), jnp.float32)
    return _run(x, mu, log_sigma, eps)


# speedup vs baseline: 2.2097x; 1.0521x over previous
import functools

import jax
import jax.numpy as jnp
from jax.experimental import pallas as pl
from jax.experimental.pallas import tpu as pltpu

_B, _T, _D, _K, _S = 4, 2048, 1024, 64, 8
_TT = 2048


def _router_body(x_ref, mu_ref, ls_ref, eps_ref, out_ref, s_acc):
    b = pl.program_id(0)
    t = pl.program_id(1)
    nt = pl.num_programs(1)

    @pl.when((b == 0) & (t == 0))
    def _init():
        s_acc[...] = jnp.zeros_like(s_acc)

    x = x_ref[0]
    mu = mu_ref[...]
    dot = jax.lax.dot_general(
        x, mu, (((1,), (1,)), ((), ())),
        preferred_element_type=jnp.float32)
    x_norm = jnp.sqrt(jnp.sum(x * x, axis=1, keepdims=True))
    mu_norm = jnp.sqrt(jnp.sum(mu * mu, axis=1))
    cos = dot / jnp.maximum(x_norm * mu_norm[None, :], 1e-8)
    m = jnp.max(cos, axis=1, keepdims=True)
    e = jnp.exp(cos - m)
    p = e / jnp.sum(e, axis=1, keepdims=True)
    partial = jnp.sum(p, axis=0, keepdims=True)
    row = jax.lax.broadcasted_iota(jnp.int32, (_B, 1), 0) == b
    s_acc[...] += jnp.where(row, partial, 0.0)

    @pl.when((b == _B - 1) & (t == nt - 1))
    def _finish():
        s = s_acc[...]
        sm = jnp.max(s, axis=1, keepdims=True)
        se = jnp.exp(s - sm)
        r = se / jnp.sum(se, axis=1, keepdims=True)
        iota_k = jax.lax.broadcasted_iota(jnp.int32, (_B, _K), 1)
        rr = r
        for j in range(_S):
            mj = jnp.max(rr, axis=1, keepdims=True)
            idxj = jnp.min(jnp.where(rr == mj, iota_k, _K), axis=1,
                           keepdims=True)
            oh = (iota_k == idxj).astype(jnp.float32)
            mu_j = jax.lax.dot_general(
                oh, mu_ref[...], (((1,), (0,)), ((), ())),
                preferred_element_type=jnp.float32,
                precision=jax.lax.Precision.HIGHEST)
            ls_j = jax.lax.dot_general(
                oh, ls_ref[...], (((1,), (0,)), ((), ())),
                preferred_element_type=jnp.float32,
                precision=jax.lax.Precision.HIGHEST)
            out_ref[:, j, :] = mu_j + jnp.exp(ls_j) * eps_ref[:, j, :]
            rr = jnp.where(oh > 0, -1.0, rr)


@functools.partial(jax.jit, static_argnames=())
def _run(x, mu, log_sigma, eps):
    nt = _T // _TT
    return pl.pallas_call(
        _router_body,
        grid=(_B, nt),
        in_specs=[
            pl.BlockSpec((1, _TT, _D), lambda b, t: (b, t, 0)),
            pl.BlockSpec((_K, _D), lambda b, t: (0, 0)),
            pl.BlockSpec((_K, _D), lambda b, t: (0, 0)),
            pl.BlockSpec((_B, _S, _D), lambda b, t: (0, 0, 0)),
        ],
        out_specs=pl.BlockSpec((_B, _S, _D), lambda b, t: (0, 0, 0)),
        out_shape=jax.ShapeDtypeStruct((_B, _S, _D), jnp.float32),
        scratch_shapes=[pltpu.VMEM((_B, _K), jnp.float32)],
        compiler_params=pltpu.CompilerParams(
            dimension_semantics=("arbitrary", "arbitrary")),
    )(x, mu, log_sigma, eps)


def kernel(x, mu, log_sigma, n_slots):
    eps = jax.random.normal(jax.random.key(42), (_B, _S, _D), jnp.float32)
    return _run(x, mu, log_sigma, eps)


# speedup vs baseline: 2.2395x; 1.0134x over previous
import functools

import jax
import jax.numpy as jnp
from jax.experimental import pallas as pl
from jax.experimental.pallas import tpu as pltpu

_B, _T, _D, _K, _S = 4, 2048, 1024, 64, 8
_TT = 2048


def _router_body(x_ref, mu_ref, ls_ref, eps_ref, out_ref, s_acc):
    b = pl.program_id(0)
    t = pl.program_id(1)
    nt = pl.num_programs(1)

    @pl.when((b == 0) & (t == 0))
    def _init():
        s_acc[...] = jnp.zeros_like(s_acc)

    x = x_ref[0]
    mu = mu_ref[...]
    dot = jax.lax.dot_general(
        x, mu, (((1,), (1,)), ((), ())),
        preferred_element_type=jnp.float32)
    x_norm = jnp.sqrt(jnp.sum(x * x, axis=1, keepdims=True))
    mu_norm = jnp.sqrt(jnp.sum(mu * mu, axis=1))
    cos = dot / jnp.maximum(x_norm * mu_norm[None, :], 1e-8)
    m = jnp.max(cos, axis=1, keepdims=True)
    e = jnp.exp(cos - m)
    p = e / jnp.sum(e, axis=1, keepdims=True)
    partial = jnp.sum(p, axis=0, keepdims=True)
    row = jax.lax.broadcasted_iota(jnp.int32, (_B, 1), 0) == b
    s_acc[...] += jnp.where(row, partial, 0.0)

    @pl.when((b == _B - 1) & (t == nt - 1))
    def _finish():
        s = s_acc[...]
        sm = jnp.max(s, axis=1, keepdims=True)
        se = jnp.exp(s - sm)
        r = se / jnp.sum(se, axis=1, keepdims=True)
        iota_k = jax.lax.broadcasted_iota(jnp.int32, (_B, _K), 1)
        rr = r
        for j in range(_S):
            mj = jnp.max(rr, axis=1, keepdims=True)
            idxj = jnp.min(jnp.where(rr == mj, iota_k, _K), axis=1,
                           keepdims=True)
            oh = (iota_k == idxj).astype(jnp.float32)
            mu_j = jax.lax.dot_general(
                oh, mu_ref[...], (((1,), (0,)), ((), ())),
                preferred_element_type=jnp.float32)
            ls_j = jax.lax.dot_general(
                oh, ls_ref[...], (((1,), (0,)), ((), ())),
                preferred_element_type=jnp.float32)
            out_ref[:, j, :] = mu_j + jnp.exp(ls_j) * eps_ref[:, j, :]
            rr = jnp.where(oh > 0, -1.0, rr)


@functools.partial(jax.jit, static_argnames=())
def _run(x, mu, log_sigma, eps):
    nt = _T // _TT
    return pl.pallas_call(
        _router_body,
        grid=(_B, nt),
        in_specs=[
            pl.BlockSpec((1, _TT, _D), lambda b, t: (b, t, 0)),
            pl.BlockSpec((_K, _D), lambda b, t: (0, 0)),
            pl.BlockSpec((_K, _D), lambda b, t: (0, 0)),
            pl.BlockSpec((_B, _S, _D), lambda b, t: (0, 0, 0)),
        ],
        out_specs=pl.BlockSpec((_B, _S, _D), lambda b, t: (0, 0, 0)),
        out_shape=jax.ShapeDtypeStruct((_B, _S, _D), jnp.float32),
        scratch_shapes=[pltpu.VMEM((_B, _K), jnp.float32)],
        compiler_params=pltpu.CompilerParams(
            dimension_semantics=("arbitrary", "arbitrary")),
    )(x, mu, log_sigma, eps)


def kernel(x, mu, log_sigma, n_slots):
    eps = jax.random.normal(jax.random.key(42), (_B, _S, _D), jnp.float32)
    return _run(x, mu, log_sigma, eps)
